# baseline (speedup 1.0000x reference)
import functools

import jax
import jax.numpy as jnp
from jax import lax
from jax.experimental import pallas as pl
from jax.experimental.pallas import tpu as pltpu
from jax.experimental.pallas import tpu_sc as plsc

CHAR_VOCAB = 256
CHAR_EMB = 16
WORD_LEN = 20
HIDDEN = 128
OUT_DIM = 64

TOKEN_BLOCK = 512
SC_CHUNK = 2560


def _make_sc_gather(n_idx):
    info = plsc.get_sparse_core_info()
    nw = info.num_cores * info.num_subcores
    per_w = n_idx // nw
    assert n_idx % nw == 0 and per_w % SC_CHUNK == 0
    steps = per_w // SC_CHUNK
    mesh = plsc.VectorSubcoreMesh(core_axis_name="c", subcore_axis_name="s")

    @functools.partial(
        pl.kernel,
        mesh=mesh,
        compiler_params=pltpu.CompilerParams(use_tc_tiling_on_sc=False),
        out_type=jax.ShapeDtypeStruct((n_idx, CHAR_EMB), jnp.float32),
        scratch_types=[
            pltpu.VMEM((SC_CHUNK,), jnp.int32),
            pltpu.VMEM((SC_CHUNK, CHAR_EMB), jnp.float32),
            pltpu.SemaphoreType.DMA,
        ],
    )
    def sc_gather(idx_hbm, table_hbm, out_hbm, idx_v, rows_v, sem):
        wid = lax.axis_index("s") * info.num_cores + lax.axis_index("c")
        w_base = wid * per_w

        def body(i, carry):
            base = w_base + i * SC_CHUNK
            pltpu.sync_copy(idx_hbm.at[pl.ds(base, SC_CHUNK)], idx_v)
            pltpu.async_copy(table_hbm.at[idx_v], rows_v, sem).wait()
            pltpu.sync_copy(rows_v, out_hbm.at[pl.ds(base, SC_CHUNK)])
            return carry

        lax.fori_loop(0, steps, body, 0)

    return sc_gather


def _mlp_kernel(x_ref, w1_ref, b1_ref, w2_ref, b2_ref, out_ref):
    x = x_ref[...].astype(jnp.bfloat16)
    acc = jnp.dot(x, w1_ref[...], preferred_element_type=jnp.float32)
    h = jax.nn.relu(acc + b1_ref[...])
    out = jnp.dot(h, w2_ref[...], preferred_element_type=jnp.float32)
    out_ref[...] = out + b2_ref[...]


def kernel(chars, emb, W1, b1, W2, b2):
    b, s, w = chars.shape
    n = b * s
    n_idx = n * w

    ce = _make_sc_gather(n_idx)(chars.reshape(n_idx), emb)
    x = ce.reshape(n, w * CHAR_EMB)

    grid = (n // TOKEN_BLOCK,)
    out = pl.pallas_call(
        _mlp_kernel,
        grid=grid,
        in_specs=[
            pl.BlockSpec((TOKEN_BLOCK, w * CHAR_EMB), lambda i: (i, 0)),
            pl.BlockSpec((w * CHAR_EMB, HIDDEN), lambda i: (0, 0)),
            pl.BlockSpec((1, HIDDEN), lambda i: (0, 0)),
            pl.BlockSpec((HIDDEN, OUT_DIM), lambda i: (0, 0)),
            pl.BlockSpec((1, OUT_DIM), lambda i: (0, 0)),
        ],
        out_specs=pl.BlockSpec((TOKEN_BLOCK, OUT_DIM), lambda i: (i, 0)),
        out_shape=jax.ShapeDtypeStruct((n, OUT_DIM), jnp.float32),
    )(x, W1.astype(jnp.bfloat16), b1.reshape(1, HIDDEN), W2,
      b2.reshape(1, OUT_DIM))

    return out.reshape(b, s, OUT_DIM)
